# baseline (speedup 1.0000x reference)
import jax
import jax.numpy as jnp
from jax import lax
from jax.experimental import pallas as pl
from jax.experimental.pallas import tpu as pltpu
from jax.experimental.pallas import tpu_sc as plsc

NC, NS = 2, 16
NW = NC * NS
B, P, D = 1024, 77, 768
ROWS_PER_W = B * P // NW
R = 44
NSTEP = ROWS_PER_W // R
LANES = 16
G = D // LANES


def _body(idx_hbm, table_hbm, pos_hbm, out_hbm, idx_v, pos_v, rows2, sem_g, sem_w):
    wid = lax.axis_index("s") * NC + lax.axis_index("c")
    base = wid * ROWS_PER_W

    pltpu.sync_copy(idx_hbm.at[pl.ds(wid * NSTEP, NSTEP)], idx_v)
    pltpu.sync_copy(pos_hbm, pos_v)

    def gather_start(s, b):
        pltpu.async_copy(table_hbm.at[idx_v.at[s]], rows2.at[b], sem_g)

    def gather_wait(b):
        pltpu.make_async_copy(table_hbm.at[idx_v.at[0]], rows2.at[b], sem_g).wait()

    def write_start(s, b):
        pltpu.async_copy(rows2.at[b], out_hbm.at[pl.ds(base + s * R, R)], sem_w)

    def write_wait(b):
        pltpu.make_async_copy(rows2.at[b], out_hbm.at[pl.ds(base, R)], sem_w).wait()

    def add_pos(s, b):
        def row(r, p):
            for g in range(G):
                sl = pl.ds(g * LANES, LANES)
                plsc.addupdate(rows2.at[b, r, sl], pos_v[p, sl])
            return lax.select(p + 1 >= P, 0, p + 1)

        lax.fori_loop(0, R, row, lax.rem(s * R, P))

    def write_sync(s, b):
        pltpu.sync_copy(rows2.at[b], out_hbm.at[pl.ds(base + s * R, R)])

    gather_start(0, 0)
    gather_wait(0)
    gather_start(1, 1)
    add_pos(0, 0)
    write_sync(0, 0)

    def pair(gi, c):
        s1 = 1 + 2 * gi
        for s, b in ((s1, 1), (s1 + 1, 0)):
            gather_wait(b)
            gather_start(s + 1, 1 - b)
            add_pos(s, b)
            write_sync(s, b)
        return c

    lax.fori_loop(0, (NSTEP - 2) // 2, pair, 0)

    gather_wait(1)
    add_pos(NSTEP - 1, 1)
    write_sync(NSTEP - 1, 1)


def kernel(input_tokens, token_table, position_table):
    idx = input_tokens.astype(jnp.int32).reshape(B * P // R, R)
    mesh = plsc.VectorSubcoreMesh(
        core_axis_name="c", subcore_axis_name="s", num_cores=NC, num_subcores=NS
    )
    out = pl.kernel(
        _body,
        out_type=jax.ShapeDtypeStruct((B * P, D), jnp.float32),
        mesh=mesh,
        compiler_params=pltpu.CompilerParams(use_tc_tiling_on_sc=False),
        scratch_types=[
            pltpu.VMEM((NSTEP, R), jnp.int32),
            pltpu.VMEM((P, D), jnp.float32),
            pltpu.VMEM((2, R, D), jnp.float32),
            pltpu.SemaphoreType.DMA,
            pltpu.SemaphoreType.DMA,
        ],
    )(idx, token_table, position_table)
    return out.reshape(B, P, D)


# speedup vs baseline: 1.1509x; 1.1509x over previous
import jax
import jax.numpy as jnp
from jax import lax
from jax.experimental import pallas as pl
from jax.experimental.pallas import tpu as pltpu
from jax.experimental.pallas import tpu_sc as plsc

NC, NS = 2, 16
NW = NC * NS
B, P, D = 1024, 77, 768
ROWS_PER_W = B * P // NW
R = 32
NSTEP = ROWS_PER_W // R
LANES = 16
G = D // LANES


def _body(idx_hbm, table_hbm, pos_hbm, out_hbm, idx_v, pos_v, buf0, buf1, sem0, sem1):
    wid = lax.axis_index("s") * NC + lax.axis_index("c")
    base = wid * ROWS_PER_W

    pltpu.sync_copy(idx_hbm.at[pl.ds(base, ROWS_PER_W + R)], idx_v)
    pltpu.sync_copy(pos_hbm, pos_v)

    def gather(s, buf, sem):
        pltpu.async_copy(table_hbm.at[idx_v.at[pl.ds(s * R, R)]], buf, sem)

    def gwait(buf, sem):
        pltpu.make_async_copy(table_hbm.at[idx_v.at[pl.ds(0, R)]], buf, sem).wait()

    def add_pos(s, buf):
        def row(r, p):
            for g in range(G):
                plsc.addupdate(
                    buf.at[r, pl.ds(g * LANES, LANES)],
                    pos_v[pl.ds(p * D + g * LANES, LANES)],
                )
            return lax.select(p + 1 >= P, 0, p + 1)

        lax.fori_loop(0, R, row, lax.rem(s * R, P))

    def write(s, buf):
        m0 = (base + s * R) // 8
        for q in range(R // 8):
            pltpu.sync_copy(buf.at[pl.ds(q * 8, 8)], out_hbm.at[m0 + q])

    gather(0, buf0, sem0)

    gwait(buf0, sem0)
    gather(1, buf1, sem1)
    add_pos(0, buf0)
    write(0, buf0)

    def pair(gi, c):
        s1 = 1 + 2 * gi
        gwait(buf1, sem1)
        gather(s1 + 1, buf0, sem0)
        add_pos(s1, buf1)
        write(s1, buf1)
        gwait(buf0, sem0)
        gather(s1 + 2, buf1, sem1)
        add_pos(s1 + 1, buf0)
        write(s1 + 1, buf0)
        return c

    lax.fori_loop(0, (NSTEP - 1) // 2, pair, 0)
    gwait(buf1, sem1)


def kernel(input_tokens, token_table, position_table):
    idx = jnp.pad(input_tokens.astype(jnp.int32).reshape(-1), (0, R))
    pos = position_table.reshape(-1)
    mesh = plsc.VectorSubcoreMesh(
        core_axis_name="c", subcore_axis_name="s", num_cores=NC, num_subcores=NS
    )
    out = pl.kernel(
        _body,
        out_type=jax.ShapeDtypeStruct((B * P // 8, 8, D), jnp.float32),
        mesh=mesh,
        scratch_types=[
            pltpu.VMEM((ROWS_PER_W + R,), jnp.int32),
            pltpu.VMEM((P * D,), jnp.float32),
            pltpu.VMEM((R, D), jnp.float32),
            pltpu.VMEM((R, D), jnp.float32),
            pltpu.SemaphoreType.DMA,
            pltpu.SemaphoreType.DMA,
        ],
    )(idx, token_table, pos)
    return out.reshape(B, P, D)


# speedup vs baseline: 1.1723x; 1.0186x over previous
import jax
import jax.numpy as jnp
from jax import lax
from jax.experimental import pallas as pl
from jax.experimental.pallas import tpu as pltpu
from jax.experimental.pallas import tpu_sc as plsc

NC, NS = 2, 16
NW = NC * NS
B, P, D = 1024, 77, 768
ROWS_PER_W = B * P // NW
R = 32
Q = R // 8
NSTEP = ROWS_PER_W // R
LANES = 16
G = D // LANES


def _body(idx_hbm, table_hbm, pos_hbm, out_hbm,
          idx_v, pos_v, buf0, buf1, g0, g1, w0, w1):
    wid = lax.axis_index("s") * NC + lax.axis_index("c")
    base = wid * ROWS_PER_W

    pltpu.sync_copy(idx_hbm.at[pl.ds(base, ROWS_PER_W + R)], idx_v)
    pltpu.sync_copy(pos_hbm, pos_v)

    def gather(s, buf, sem):
        for q in range(Q):
            pltpu.async_copy(
                table_hbm.at[idx_v.at[pl.ds(s * R + q * 8, 8)]], buf.at[q], sem
            )

    def gwait(buf, sem):
        for q in range(Q):
            pltpu.make_async_copy(
                table_hbm.at[idx_v.at[pl.ds(0, 8)]], buf.at[q], sem
            ).wait()

    def add_pos(s, buf):
        def row(r, p):
            q = lax.shift_right_logical(r, 3)
            rr = lax.bitwise_and(r, 7)
            for g in range(G):
                plsc.addupdate(
                    buf.at[q, rr, pl.ds(g * LANES, LANES)],
                    pos_v[pl.ds(p * D + g * LANES, LANES)],
                )
            return lax.select(p + 1 >= P, 0, p + 1)

        lax.fori_loop(0, R, row, lax.rem(s * R, P))

    def wstart(s, buf, sem):
        m0 = (base + s * R) // 8
        pltpu.async_copy(buf, out_hbm.at[pl.ds(m0, Q)], sem)

    def wwait(buf, sem):
        pltpu.make_async_copy(buf, out_hbm.at[pl.ds(0, Q)], sem).wait()

    gather(0, buf0, g0)

    gwait(buf0, g0)
    gather(1, buf1, g1)
    add_pos(0, buf0)
    wstart(0, buf0, w0)

    def pair(gi, c):
        s1 = 1 + 2 * gi
        gwait(buf1, g1)
        wwait(buf0, w0)
        gather(s1 + 1, buf0, g0)
        add_pos(s1, buf1)
        wstart(s1, buf1, w1)
        gwait(buf0, g0)
        wwait(buf1, w1)
        gather(s1 + 2, buf1, g1)
        add_pos(s1 + 1, buf0)
        wstart(s1 + 1, buf0, w0)
        return c

    lax.fori_loop(0, (NSTEP - 1) // 2, pair, 0)

    gwait(buf1, g1)
    wwait(buf0, w0)


def kernel(input_tokens, token_table, position_table):
    idx = jnp.pad(input_tokens.astype(jnp.int32).reshape(-1), (0, R))
    pos = position_table.reshape(-1)
    mesh = plsc.VectorSubcoreMesh(
        core_axis_name="c", subcore_axis_name="s", num_cores=NC, num_subcores=NS
    )
    out = pl.kernel(
        _body,
        out_type=jax.ShapeDtypeStruct((B * P // 8, 8, D), jnp.float32),
        mesh=mesh,
        scratch_types=[
            pltpu.VMEM((ROWS_PER_W + R,), jnp.int32),
            pltpu.VMEM((P * D,), jnp.float32),
            pltpu.VMEM((Q, 8, D), jnp.float32),
            pltpu.VMEM((Q, 8, D), jnp.float32),
            pltpu.SemaphoreType.DMA,
            pltpu.SemaphoreType.DMA,
            pltpu.SemaphoreType.DMA,
            pltpu.SemaphoreType.DMA,
        ],
    )(idx, token_table, pos)
    return out.reshape(B, P, D)


# speedup vs baseline: 1.2124x; 1.0342x over previous
import jax
import jax.numpy as jnp
from jax import lax
from jax.experimental import pallas as pl
from jax.experimental.pallas import tpu as pltpu
from jax.experimental.pallas import tpu_sc as plsc

NC, NS = 2, 16
NW = NC * NS
B, P, D = 1024, 77, 768
ROWS_PER_W = B * P // NW
R = 16
Q = R // 8
NSTEP = ROWS_PER_W // R
PE = P + R - 1
LANES = 16
G = D // LANES


def _body(idx_hbm, table_hbm, pos_hbm, out_hbm,
          idx_v, pos_v, buf0, buf1, g0, g1, w0, w1):
    wid = lax.axis_index("s") * NC + lax.axis_index("c")
    base = wid * ROWS_PER_W

    pltpu.sync_copy(idx_hbm.at[pl.ds(base, ROWS_PER_W + R)], idx_v)
    pltpu.sync_copy(pos_hbm, pos_v)

    def gather(s, buf, sem):
        for q in range(Q):
            pltpu.async_copy(
                table_hbm.at[idx_v.at[pl.ds(s * R + q * 8, 8)]], buf.at[q], sem
            )

    def gwait(buf, sem):
        for q in range(Q):
            pltpu.make_async_copy(
                table_hbm.at[idx_v.at[pl.ds(0, 8)]], buf.at[q], sem
            ).wait()

    def add_pos(s, buf):
        p0d = lax.rem(s * R, P) * D
        K = 8
        for r in range(R):
            for g0 in range(0, G, K):
                vs = [
                    pos_v[pl.ds(p0d + r * D + (g0 + k) * LANES, LANES)]
                    for k in range(K)
                ]
                for k in range(K):
                    plsc.addupdate(
                        buf.at[r // 8, r % 8, pl.ds((g0 + k) * LANES, LANES)], vs[k]
                    )

    def wstart(s, buf, sem):
        m0 = (base + s * R) // 8
        pltpu.async_copy(buf, out_hbm.at[pl.ds(m0, Q)], sem)

    def wwait(buf, sem):
        pltpu.make_async_copy(buf, out_hbm.at[pl.ds(0, Q)], sem).wait()

    gather(0, buf0, g0)

    gwait(buf0, g0)
    gather(1, buf1, g1)
    add_pos(0, buf0)
    wstart(0, buf0, w0)

    def pair(gi, c):
        s1 = 1 + 2 * gi
        gwait(buf1, g1)
        wwait(buf0, w0)
        gather(s1 + 1, buf0, g0)
        add_pos(s1, buf1)
        wstart(s1, buf1, w1)
        gwait(buf0, g0)
        wwait(buf1, w1)
        gather(s1 + 2, buf1, g1)
        add_pos(s1 + 1, buf0)
        wstart(s1 + 1, buf0, w0)
        return c

    lax.fori_loop(0, (NSTEP - 2) // 2, pair, 0)

    gwait(buf1, g1)
    wwait(buf0, w0)
    add_pos(NSTEP - 1, buf1)
    wstart(NSTEP - 1, buf1, w1)
    wwait(buf1, w1)


def kernel(input_tokens, token_table, position_table):
    idx = jnp.pad(input_tokens.astype(jnp.int32).reshape(-1), (0, R))
    pos = jnp.concatenate([position_table, position_table[: R - 1]]).reshape(-1)
    mesh = plsc.VectorSubcoreMesh(
        core_axis_name="c", subcore_axis_name="s", num_cores=NC, num_subcores=NS
    )
    out = pl.kernel(
        _body,
        out_type=jax.ShapeDtypeStruct((B * P // 8, 8, D), jnp.float32),
        mesh=mesh,
        scratch_types=[
            pltpu.VMEM((ROWS_PER_W + R,), jnp.int32),
            pltpu.VMEM((PE * D,), jnp.float32),
            pltpu.VMEM((Q, 8, D), jnp.float32),
            pltpu.VMEM((Q, 8, D), jnp.float32),
            pltpu.SemaphoreType.DMA,
            pltpu.SemaphoreType.DMA,
            pltpu.SemaphoreType.DMA,
            pltpu.SemaphoreType.DMA,
        ],
    )(idx, token_table, pos)
    return out.reshape(B, P, D)


# speedup vs baseline: 1.3030x; 1.0747x over previous
import jax
import jax.numpy as jnp
from jax import lax
from jax.experimental import pallas as pl
from jax.experimental.pallas import tpu as pltpu
from jax.experimental.pallas import tpu_sc as plsc

NC, NS = 2, 16
NW = NC * NS
B, P, D = 1024, 77, 768
ROWS_PER_W = B * P // NW
R = 16
Q = R // 8
NSTEP = ROWS_PER_W // R
PE = P + R - 1
LANES = 16
G = D // LANES


def _body(idx_hbm, table_hbm, pos_hbm, out_hbm,
          idx_v, pos_v, buf0, buf1, g0, g1, w0, w1):
    wid = lax.axis_index("s") * NC + lax.axis_index("c")
    base = wid * ROWS_PER_W

    pltpu.sync_copy(idx_hbm.at[pl.ds(base, ROWS_PER_W + R)], idx_v)
    pltpu.sync_copy(pos_hbm, pos_v)

    def gather(s, buf, sem):
        for q in range(Q):
            pltpu.async_copy(
                table_hbm.at[idx_v.at[pl.ds(s * R + q * 8, 8)]], buf.at[q], sem
            )

    def gwait(buf, sem):
        for q in range(Q):
            pltpu.make_async_copy(
                table_hbm.at[idx_v.at[pl.ds(0, 8)]], buf.at[q], sem
            ).wait()

    def add_pos(s, buf):
        del s, buf

    def wstart(s, buf, sem):
        m0 = (base + s * R) // 8
        pltpu.async_copy(buf, out_hbm.at[pl.ds(m0, Q)], sem)

    def wwait(buf, sem):
        pltpu.make_async_copy(buf, out_hbm.at[pl.ds(0, Q)], sem).wait()

    gather(0, buf0, g0)

    gwait(buf0, g0)
    gather(1, buf1, g1)
    add_pos(0, buf0)
    wstart(0, buf0, w0)

    def pair(gi, c):
        s1 = 1 + 2 * gi
        gwait(buf1, g1)
        wwait(buf0, w0)
        gather(s1 + 1, buf0, g0)
        add_pos(s1, buf1)
        wstart(s1, buf1, w1)
        gwait(buf0, g0)
        wwait(buf1, w1)
        gather(s1 + 2, buf1, g1)
        add_pos(s1 + 1, buf0)
        wstart(s1 + 1, buf0, w0)
        return c

    lax.fori_loop(0, (NSTEP - 2) // 2, pair, 0)

    gwait(buf1, g1)
    wwait(buf0, w0)
    add_pos(NSTEP - 1, buf1)
    wstart(NSTEP - 1, buf1, w1)
    wwait(buf1, w1)


def kernel(input_tokens, token_table, position_table):
    idx = jnp.pad(input_tokens.astype(jnp.int32).reshape(-1), (0, R))
    pos = jnp.concatenate([position_table, position_table[: R - 1]]).reshape(-1)
    mesh = plsc.VectorSubcoreMesh(
        core_axis_name="c", subcore_axis_name="s", num_cores=NC, num_subcores=NS
    )
    out = pl.kernel(
        _body,
        out_type=jax.ShapeDtypeStruct((B * P // 8, 8, D), jnp.float32),
        mesh=mesh,
        scratch_types=[
            pltpu.VMEM((ROWS_PER_W + R,), jnp.int32),
            pltpu.VMEM((PE * D,), jnp.float32),
            pltpu.VMEM((Q, 8, D), jnp.float32),
            pltpu.VMEM((Q, 8, D), jnp.float32),
            pltpu.SemaphoreType.DMA,
            pltpu.SemaphoreType.DMA,
            pltpu.SemaphoreType.DMA,
            pltpu.SemaphoreType.DMA,
        ],
    )(idx, token_table, pos)
    return out.reshape(B, P, D) + position_table[None]
